# Initial kernel scaffold; baseline (speedup 1.0000x reference)
import jax
import jax.numpy as jnp
from jax.experimental import pallas as pl


def kernel(x, edge_index, batch, W1, b1, W2, b2, W3, b3, gate_W, gate_b, Wr, br):
    raise NotImplementedError("write your pallas kernel here")


# speedup vs baseline: 12.0563x; 12.0563x over previous
import functools

import jax
import jax.numpy as jnp
from jax import lax
from jax.experimental import pallas as pl
from jax.experimental.pallas import tpu as pltpu
from jax.experimental.pallas import tpu_sc as plsc

N = 10000
E = 320000
IN = 128
HD = 128
G = 64
F = 3

NC = 2
NS = 16
NW = NC * NS
EPT = E // NW
C = 80
NCHUNK = EPT // C
NPAD = 10240
RPT = NPAD // NS

_mesh = plsc.VectorSubcoreMesh(core_axis_name="c", subcore_axis_name="s")


@functools.partial(
    pl.kernel,
    out_type=jax.ShapeDtypeStruct((NW * N,), jnp.float32),
    mesh=_mesh,
    scratch_types=[
        pltpu.VMEM((N,), jnp.float32),
        pltpu.VMEM((EPT,), jnp.int32),
    ],
    compiler_params=pltpu.CompilerParams(needs_layout_passes=False),
)
def _deg_kernel(dst_hbm, out_hbm, deg_v, idx_v):
    cid = lax.axis_index("c")
    sid = lax.axis_index("s")
    wid = cid * NS + sid

    zero16 = jnp.zeros((16,), jnp.float32)

    def zbody(i, carry):
        deg_v[pl.ds(i * 16, 16)] = zero16
        return carry

    lax.fori_loop(0, N // 16, zbody, 0)

    pltpu.sync_copy(dst_hbm.at[pl.ds(wid * EPT, EPT)], idx_v)

    ones16 = jnp.ones((16,), jnp.float32)

    def body(i, carry):
        idx16 = idx_v[pl.ds(i * 16, 16)]
        plsc.addupdate_scatter(deg_v, [idx16], ones16)
        return carry

    lax.fori_loop(0, EPT // 16, body, 0)
    pltpu.sync_copy(deg_v, out_hbm.at[pl.ds(wid * N, N)])


@functools.partial(
    pl.kernel,
    out_type=jax.ShapeDtypeStruct((NC, NPAD, HD), jnp.float32),
    mesh=_mesh,
    scratch_types=[
        pltpu.VMEM_SHARED((NPAD, HD), jnp.float32),
        pltpu.VMEM((C,), jnp.int32),
        pltpu.VMEM((C,), jnp.int32),
        pltpu.VMEM((C, HD), jnp.float32),
        pltpu.SemaphoreType.DMA,
    ],
)
def _edge_kernel(xs_hbm, src_hbm, dst_hbm, zeros_hbm, out_hbm,
                 acc_s, src_v, dst_v, rows_v, sem):
    cid = lax.axis_index("c")
    sid = lax.axis_index("s")
    wid = cid * NS + sid

    row0 = sid * RPT
    pltpu.sync_copy(zeros_hbm.at[pl.ds(row0, RPT)], acc_s.at[pl.ds(row0, RPT)])
    plsc.subcore_barrier()

    def body(j, carry):
        base = wid * EPT + j * C
        pltpu.sync_copy(src_hbm.at[pl.ds(base, C)], src_v)
        pltpu.sync_copy(dst_hbm.at[pl.ds(base, C)], dst_v)
        pltpu.async_copy(xs_hbm.at[src_v], rows_v, sem).wait()
        pltpu.sync_copy(rows_v, acc_s.at[dst_v], add=True)
        return carry

    lax.fori_loop(0, NCHUNK, body, 0)
    plsc.subcore_barrier()
    pltpu.sync_copy(acc_s.at[pl.ds(row0, RPT)], out_hbm.at[cid, pl.ds(row0, RPT)])


def _prep_body(degp_ref, x_ref, W1_ref, dinv_ref, xs_ref):
    ones = jnp.ones((NW, 1), jnp.float32)
    deg = lax.dot_general(degp_ref[...], ones, (((0,), (0,)), ((), ())))
    dinv = lax.rsqrt(deg + 1.0)
    xw = jnp.dot(x_ref[...], W1_ref[...], preferred_element_type=jnp.float32)
    dinv_ref[...] = dinv
    xs_ref[...] = xw * dinv


_tc_prep = pl.pallas_call(
    _prep_body,
    out_shape=[
        jax.ShapeDtypeStruct((N, 1), jnp.float32),
        jax.ShapeDtypeStruct((N, HD), jnp.float32),
    ],
)


def _layer_body(acc_ref, xs_ref, dinv_ref, b_ref, W_ref, out_ref):
    dinv = dinv_ref[...]
    acc = (acc_ref[0] + acc_ref[1])[:N]
    h = dinv * (acc + xs_ref[...]) + b_ref[...]
    h = jnp.maximum(h, 0.0)
    out_ref[...] = jnp.dot(h, W_ref[...],
                           preferred_element_type=jnp.float32) * dinv


_tc_layer = pl.pallas_call(
    _layer_body,
    out_shape=jax.ShapeDtypeStruct((N, HD), jnp.float32),
)


def _final_body(acc_ref, xs_ref, dinv_ref, b_ref, batch_ref, gW_ref, gb_ref,
                Wr_ref, br_ref, out_ref):
    dinv = dinv_ref[...]
    acc = (acc_ref[0] + acc_ref[1])[:N]
    h = dinv * (acc + xs_ref[...]) + b_ref[...]
    gate = jnp.dot(h, gW_ref[...], preferred_element_type=jnp.float32)
    gate = gate + gb_ref[...]

    gid = lax.broadcasted_iota(jnp.int32, (N, G), 1)
    M = batch_ref[...] == gid
    Mf = M.astype(jnp.float32)
    neg = jnp.float32(-1e30)
    gmax = jnp.max(jnp.where(M, gate, neg), axis=0, keepdims=True)
    gmax_n = jnp.sum(Mf * gmax, axis=1, keepdims=True)
    gexp = jnp.exp(gate - gmax_n)
    gsum = jnp.sum(Mf * gexp, axis=0, keepdims=True)
    gsum_n = jnp.sum(Mf * gsum, axis=1, keepdims=True)
    alpha = gexp / gsum_n
    pooled = lax.dot_general(Mf, alpha * h, (((0,), (0,)), ((), ())))
    out_ref[...] = jnp.tanh(
        jnp.dot(pooled, Wr_ref[...], preferred_element_type=jnp.float32)
        + br_ref[...])


_tc_final = pl.pallas_call(
    _final_body,
    out_shape=jax.ShapeDtypeStruct((G, F), jnp.float32),
)


def kernel(x, edge_index, batch, W1, b1, W2, b2, W3, b3,
           gate_W, gate_b, Wr, br):
    src = edge_index[0]
    dst = edge_index[1]
    zeros = jnp.zeros((NPAD, HD), jnp.float32)

    degp = _deg_kernel(dst).reshape(NW, N)
    dinv, xs1 = _tc_prep(degp, x, W1)
    acc1 = _edge_kernel(xs1, src, dst, zeros)
    xs2 = _tc_layer(acc1, xs1, dinv, b1.reshape(1, HD), W2)
    acc2 = _edge_kernel(xs2, src, dst, zeros)
    xs3 = _tc_layer(acc2, xs2, dinv, b2.reshape(1, HD), W3)
    acc3 = _edge_kernel(xs3, src, dst, zeros)
    out = _tc_final(acc3, xs3, dinv, b3.reshape(1, HD),
                    batch.reshape(N, 1), gate_W, gate_b.reshape(1, 1),
                    Wr, br.reshape(1, F))
    return out
